# Initial kernel scaffold; baseline (speedup 1.0000x reference)
import jax
import jax.numpy as jnp
from jax.experimental import pallas as pl


def kernel(x, edge_index, enc_W, enc_b, pre_W, pre_b, post_W, post_b, lin_W, lin_b, bn_gamma, bn_beta):
    raise NotImplementedError("write your pallas kernel here")


# speedup vs baseline: 37.6873x; 37.6873x over previous
import functools
import math

import jax
import jax.numpy as jnp
from jax.experimental import pallas as pl

_N = 50000
_E = 800000
_D_IN = 128
_EMB = 48
_T = 4
_F = 12
_L = 4
_AVG_LOG = float(math.log(17.0))
_BR = 1000


def _mm_body(x_ref, w_ref, b_ref, o_ref):
    o_ref[...] = (
        jnp.dot(x_ref[...], w_ref[...], preferred_element_type=jnp.float32)
        + b_ref[...]
    )


def _mm(x, w, b):
    n, k = x.shape
    m = w.shape[1]
    return pl.pallas_call(
        _mm_body,
        grid=(n // _BR,),
        in_specs=[
            pl.BlockSpec((_BR, k), lambda i: (i, 0)),
            pl.BlockSpec((k, m), lambda i: (0, 0)),
            pl.BlockSpec((1, m), lambda i: (0, 0)),
        ],
        out_specs=pl.BlockSpec((_BR, m), lambda i: (i, 0)),
        out_shape=jax.ShapeDtypeStruct((n, m), jnp.float32),
    )(x, w, b.reshape(1, m))


def _blockdiag(w):
    t, k, f = w.shape
    out = jnp.zeros((t, k, t, f), jnp.float32)
    for i in range(t):
        out = out.at[i, :, i, :].set(w[i])
    return out.reshape(t * k, t * f)


def kernel(x, edge_index, enc_W, enc_b, pre_W, pre_b, post_W, post_b,
           lin_W, lin_b, bn_gamma, bn_beta):
    src, dst = edge_index[0], edge_index[1]
    deg = jnp.zeros((_N,), jnp.float32).at[dst].add(1.0)
    deg_c = jnp.maximum(deg, 1.0)
    log_deg = jnp.log(deg_c + 1.0)
    has = (deg > 0)[:, None]
    amp = (log_deg / _AVG_LOG)[:, None]
    att = (_AVG_LOG / log_deg)[:, None]
    dcv = deg_c[:, None]

    h = _mm(x, enc_W, enc_b)
    for l in range(_L):
        Wd = _blockdiag(pre_W[l, :, :_F, :])
        Ws = _blockdiag(pre_W[l, :, _F:, :])
        A = _mm(h, Wd, pre_b[l].reshape(_EMB))
        B = _mm(h, Ws, jnp.zeros((_EMB,), jnp.float32))
        Bs = B[src]
        S1 = jax.ops.segment_sum(Bs, dst, num_segments=_N)
        S2 = jax.ops.segment_sum(Bs * Bs, dst, num_segments=_N)
        Mx = jax.ops.segment_max(Bs, dst, num_segments=_N)
        Mn = jax.ops.segment_min(Bs, dst, num_segments=_N)
        mean = jnp.where(has, A + S1 / dcv, 0.0)
        std = jnp.sqrt(jax.nn.relu(S2 / dcv - (S1 / dcv) ** 2) + 1e-5)
        mx = jnp.where(has, A + Mx, 0.0)
        mn = jnp.where(has, A + Mn, 0.0)
        feat = jnp.concatenate([
            h, mean, mx, mn, std,
            mean * amp, mx * amp, mn * amp, std * amp,
            mean * att, mx * att, mn * att, std * att,
        ], axis=1)
        Wp = _blockdiag(post_W[l])
        Wp = Wp.reshape(_T, 13, _F, _EMB).transpose(1, 0, 2, 3).reshape(624, _EMB)
        Wfull = Wp @ lin_W[l]
        bfull = post_b[l].reshape(_EMB) @ lin_W[l] + lin_b[l]
        out = _mm(feat, Wfull, bfull)
        mu = out.mean(axis=0)
        var = out.var(axis=0)
        out = (out - mu) / jnp.sqrt(var + 1e-5) * bn_gamma[l] + bn_beta[l]
        h = jax.nn.relu(out) + h
    return h


# speedup vs baseline: 155.0347x; 4.1137x over previous
import functools
import math

import jax
import jax.numpy as jnp
from jax import lax
from jax.experimental import pallas as pl
from jax.experimental.pallas import tpu as pltpu
from jax.experimental.pallas import tpu_sc as plsc

_N = 50000
_E = 800000
_D_IN = 128
_EMB = 48
_T = 4
_F = 12
_L = 4
_AVG_LOG = float(math.log(17.0))
_BR = 1000

_NC = 2
_NS = 16
_NW = _NC * _NS
_NPT = 1600
_NPAD = _NW * _NPT
_CAP = 30720
_CH = 4000
_NCH = _E // _CH
_GC = 128

_mesh = plsc.VectorSubcoreMesh(core_axis_name="c", subcore_axis_name="s")
_sc_params = pltpu.CompilerParams(needs_layout_passes=False,
                                  use_tc_tiling_on_sc=False)

_NEG = float("-inf")
_POS = float("inf")
_RB = 1


def _wid():
    return lax.axis_index("s") * _NC + lax.axis_index("c")


@functools.partial(
    pl.kernel,
    out_type=(
        jax.ShapeDtypeStruct((_NPAD, 16), jnp.int32),
        jax.ShapeDtypeStruct((_NW * _CAP,), jnp.int32),
    ),
    mesh=_mesh,
    scratch_types=[
        pltpu.VMEM((_CH,), jnp.int32),
        pltpu.VMEM((_CH,), jnp.int32),
        pltpu.VMEM((_CAP,), jnp.int32),
        pltpu.VMEM((_CAP,), jnp.int32),
        pltpu.VMEM((_NPT, 16), jnp.int32),
        pltpu.VMEM((_CAP,), jnp.int32),
    ],
    compiler_params=_sc_params,
)
def _sc_group(edge_ref, deg16_ref, srt_ref, dbuf, sbuf, kdst, ksrc, h16, sortl):
    t = _wid()
    lo = t * _NPT
    hi = lo + _NPT
    zeros = jnp.zeros((16,), jnp.int32)
    lane = lax.iota(jnp.int32, 16)

    def _z(i, _):
        h16[i, pl.ds(0, 16)] = zeros
        return 0
    lax.fori_loop(0, _NPT, _z, 0)

    def _z2(i, _):
        sortl[pl.ds(i * 16, 16)] = zeros
        return 0
    lax.fori_loop(0, _CAP // 16, _z2, 0)

    def _chunk(c, cnt):
        pltpu.sync_copy(edge_ref.at[1, pl.ds(c * _CH, _CH)], dbuf)
        pltpu.sync_copy(edge_ref.at[0, pl.ds(c * _CH, _CH)], sbuf)

        def _vec(v, cnt):
            d = dbuf[pl.ds(v * 16, 16)]
            m = (d >= lo) & (d < hi)
            rel = d - lo
            s = sbuf[pl.ds(v * 16, 16)]
            w = jnp.minimum(cnt, _CAP - 16)
            rank = plsc.cumsum(m.astype(jnp.int32))
            pos = jnp.clip(w + rank - 1, 0, _CAP - 1)
            plsc.store_scatter(kdst, [pos], rel, mask=m)
            plsc.store_scatter(ksrc, [pos], s, mask=m)
            return cnt + rank[15]
        return lax.fori_loop(0, _CH // 16, _vec, cnt)

    cnt = lax.fori_loop(0, _NCH, _chunk, jnp.int32(0))
    cnt = jnp.minimum(cnt, _CAP)
    nv = (cnt + 15) // 16

    def _hist(v, _):
        m = (v * 16 + lane) < cnt
        d = jnp.clip(kdst[pl.ds(v * 16, 16)], 0, _NPT - 1)
        c = plsc.load_gather(h16, [d, lane], mask=m)
        plsc.store_scatter(h16, [d, lane], c + 1, mask=m)
        return 0
    lax.fori_loop(0, nv, _hist, 0)

    pltpu.sync_copy(h16, deg16_ref.at[pl.ds(t * _NPT, _NPT)])

    def _cum(i, base):
        v = h16[i, pl.ds(0, 16)]
        inc = plsc.cumsum(v)
        h16[i, pl.ds(0, 16)] = inc - v + base
        return base + inc[15]
    lax.fori_loop(0, _NPT, _cum, jnp.int32(0))

    def _place(v, _):
        m = (v * 16 + lane) < cnt
        d = jnp.clip(kdst[pl.ds(v * 16, 16)], 0, _NPT - 1)
        s = ksrc[pl.ds(v * 16, 16)]
        p = plsc.load_gather(h16, [d, lane], mask=m)
        plsc.store_scatter(sortl, [jnp.clip(p, 0, _CAP - 1)], s, mask=m)
        plsc.store_scatter(h16, [d, lane], p + 1, mask=m)
        return 0
    lax.fori_loop(0, nv, _place, 0)

    pltpu.sync_copy(sortl, srt_ref.at[pl.ds(t * _CAP, _CAP)])


@functools.partial(
    pl.kernel,
    out_type=jax.ShapeDtypeStruct((_NPAD, 4 * _EMB), jnp.float32),
    mesh=_mesh,
    scratch_types=[
        pltpu.VMEM((_NPT + 16,), jnp.int32),
        pltpu.VMEM((_CAP,), jnp.int32),
        pltpu.VMEM((2, _GC), jnp.int32),
        pltpu.VMEM((2, _GC, _EMB), jnp.float32),
        pltpu.VMEM((64, 4 * _EMB), jnp.float32),
        pltpu.SemaphoreType.DMA,
    ],
    compiler_params=_sc_params,
)
def _sc_fold(b_ref, deg_ref, srt_ref, stats_ref, degw, srt, idxb, rows,
             outb, sem):
    t = _wid()
    zf = jnp.zeros((16,), jnp.float32)
    neg = jnp.full((16,), _NEG, jnp.float32)
    pos = jnp.full((16,), _POS, jnp.float32)

    pltpu.sync_copy(deg_ref.at[pl.ds(t * _NPT, _NPT)], degw.at[pl.ds(0, _NPT)])
    pltpu.sync_copy(srt_ref.at[pl.ds(t * _CAP, _CAP)], srt)

    def _deg(n):
        return degw[pl.ds(jnp.minimum(n, _NPT - 1), 16)][0]

    def _sum(i, acc):
        return acc + degw[pl.ds(i * 16, 16)]
    tot = lax.fori_loop(0, _NPT // 16, _sum, jnp.zeros((16,), jnp.int32))
    count = jnp.minimum(plsc.cumsum(tot)[15], _CAP)
    nch = (count + _GC - 1) // _GC

    def _fire(c, k):
        def _cp(i, _):
            idxb[k, pl.ds(i * 16, 16)] = srt[pl.ds(c * _GC + i * 16, 16)]
            return 0
        lax.fori_loop(0, _GC // 16, _cp, 0)
        pltpu.async_copy(b_ref.at[idxb.at[k]], rows.at[k], sem).wait()

    def _flush_adv(state):
        n, r, s10, s11, s12, q0, q1, q2, x0, x1, x2, m0, m1, m2 = state
        row = n & 63
        outb[row, pl.ds(0, 16)] = s10
        outb[row, pl.ds(16, 16)] = s11
        outb[row, pl.ds(32, 16)] = s12
        outb[row, pl.ds(48, 16)] = q0
        outb[row, pl.ds(64, 16)] = q1
        outb[row, pl.ds(80, 16)] = q2
        outb[row, pl.ds(96, 16)] = x0
        outb[row, pl.ds(112, 16)] = x1
        outb[row, pl.ds(128, 16)] = x2
        outb[row, pl.ds(144, 16)] = m0
        outb[row, pl.ds(160, 16)] = m1
        outb[row, pl.ds(176, 16)] = m2

        @pl.when(row == 63)
        def _():
            pltpu.sync_copy(outb, stats_ref.at[pl.ds(t * _NPT + (n - 63), 64)])

        n = n + 1
        r = jnp.where(n < _NPT, _deg(n), jnp.int32(1))
        return (n, r, zf, zf, zf, zf, zf, zf, neg, neg, neg, pos, pos, pos)

    def _skip_empty(state):
        return lax.while_loop(
            lambda st: (st[1] == 0) & (st[0] < _NPT), _flush_adv, state)

    state = (jnp.int32(0), _deg(jnp.int32(0)), zf, zf, zf, zf, zf, zf,
             neg, neg, neg, pos, pos, pos)
    state = _skip_empty(state)

    def _chunkpair(cb, state):
        for k in range(2):
            c = cb * 2 + k

            @pl.when(c < nch)
            def _():
                _fire(c, k)

            def _edge(i, st):
                n, r, s10, s11, s12, q0, q1, q2, x0, x1, x2, m0, m1, m2 = st
                v0 = rows[k, i, pl.ds(0, 16)]
                v1 = rows[k, i, pl.ds(16, 16)]
                v2 = rows[k, i, pl.ds(32, 16)]
                st = (n, r - 1,
                      s10 + v0, s11 + v1, s12 + v2,
                      q0 + v0 * v0, q1 + v1 * v1, q2 + v2 * v2,
                      jnp.maximum(x0, v0), jnp.maximum(x1, v1),
                      jnp.maximum(x2, v2),
                      jnp.minimum(m0, v0), jnp.minimum(m1, v1),
                      jnp.minimum(m2, v2))
                return _skip_empty(st)

            ne = jnp.clip(count - c * _GC, 0, _GC)
            state = lax.fori_loop(0, ne, _edge, state)
        return state

    lax.fori_loop(0, (nch + 1) // 2, _chunkpair, state)


def _mm_body(x_ref, w_ref, b_ref, o_ref):
    o_ref[...] = (
        jnp.dot(x_ref[...], w_ref[...], preferred_element_type=jnp.float32)
        + b_ref[...]
    )


def _mm(x, w, b):
    n, k = x.shape
    m = w.shape[1]
    return pl.pallas_call(
        _mm_body,
        grid=(n // _BR,),
        in_specs=[
            pl.BlockSpec((_BR, k), lambda i: (i, 0)),
            pl.BlockSpec((k, m), lambda i: (0, 0)),
            pl.BlockSpec((1, m), lambda i: (0, 0)),
        ],
        out_specs=pl.BlockSpec((_BR, m), lambda i: (i, 0)),
        out_shape=jax.ShapeDtypeStruct((n, m), jnp.float32),
    )(x, w, b.reshape(1, m))


def _blockdiag(w):
    t, k, f = w.shape
    out = jnp.zeros((t, k, t, f), jnp.float32)
    for i in range(t):
        out = out.at[i, :, i, :].set(w[i])
    return out.reshape(t * k, t * f)


def kernel(x, edge_index, enc_W, enc_b, pre_W, pre_b, post_W, post_b,
           lin_W, lin_b, bn_gamma, bn_beta):
    deg16, srt = _sc_group(edge_index)
    deg_i = deg16.sum(axis=1)
    deg = deg_i[:_N].astype(jnp.float32)
    deg_c = jnp.maximum(deg, 1.0)
    log_deg = jnp.log(deg_c + 1.0)
    has = (deg > 0)[:, None]
    amp = (log_deg / _AVG_LOG)[:, None]
    att = (_AVG_LOG / log_deg)[:, None]
    dcv = deg_c[:, None]

    h = _mm(x, enc_W, enc_b)
    for l in range(_L):
        Wd = _blockdiag(pre_W[l, :, :_F, :])
        Ws = _blockdiag(pre_W[l, :, _F:, :])
        A = _mm(h, Wd, pre_b[l].reshape(_EMB))
        B = _mm(h, Ws, jnp.zeros((_EMB,), jnp.float32))
        stats = _sc_fold(B, deg_i, srt)[:_N]
        S1 = stats[:, 0:48]
        S2 = stats[:, 48:96]
        Mx = stats[:, 96:144]
        Mn = stats[:, 144:192]
        mean = jnp.where(has, A + S1 / dcv, 0.0)
        std = jnp.sqrt(jax.nn.relu(S2 / dcv - (S1 / dcv) ** 2) + 1e-5)
        mx = jnp.where(has, A + Mx, 0.0)
        mn = jnp.where(has, A + Mn, 0.0)
        feat = jnp.concatenate([
            h, mean, mx, mn, std,
            mean * amp, mx * amp, mn * amp, std * amp,
            mean * att, mx * att, mn * att, std * att,
        ], axis=1)
        Wp = _blockdiag(post_W[l])
        Wp = Wp.reshape(_T, 13, _F, _EMB).transpose(1, 0, 2, 3).reshape(624, _EMB)
        Wfull = Wp @ lin_W[l]
        bfull = post_b[l].reshape(_EMB) @ lin_W[l] + lin_b[l]
        out = _mm(feat, Wfull, bfull)
        mu = out.mean(axis=0)
        var = out.var(axis=0)
        out = (out - mu) / jnp.sqrt(var + 1e-5) * bn_gamma[l] + bn_beta[l]
        h = jax.nn.relu(out) + h
    return h


# speedup vs baseline: 175.2797x; 1.1306x over previous
import functools
import math

import jax
import jax.numpy as jnp
from jax import lax
from jax.experimental import pallas as pl
from jax.experimental.pallas import tpu as pltpu
from jax.experimental.pallas import tpu_sc as plsc

_N = 50000
_E = 800000
_D_IN = 128
_EMB = 48
_T = 4
_F = 12
_L = 4
_AVG_LOG = float(math.log(17.0))
_BR = 1000

_NC = 2
_NS = 16
_NW = _NC * _NS
_NPT = 1600
_NPAD = _NW * _NPT
_CAP = 30720
_CH = 2000
_NCH = _E // _CH
_GC = 128

_mesh = plsc.VectorSubcoreMesh(core_axis_name="c", subcore_axis_name="s")
_sc_params = pltpu.CompilerParams(needs_layout_passes=False,
                                  use_tc_tiling_on_sc=False)

_NEG = float("-inf")
_POS = float("inf")
_RB = 1


def _wid():
    return lax.axis_index("s") * _NC + lax.axis_index("c")


@functools.partial(
    pl.kernel,
    out_type=(
        jax.ShapeDtypeStruct((_NPAD, 16), jnp.int32),
        jax.ShapeDtypeStruct((_NW * _CAP,), jnp.int32),
    ),
    mesh=_mesh,
    scratch_types=[
        pltpu.VMEM((2, 2, _CH), jnp.int32),
        pltpu.VMEM((_CAP,), jnp.int32),
        pltpu.VMEM((_CAP,), jnp.int32),
        pltpu.VMEM((_NPT, 16), jnp.int32),
        pltpu.VMEM((_CAP,), jnp.int32),
        pltpu.SemaphoreType.DMA,
    ],
    compiler_params=_sc_params,
)
def _sc_group(edge_ref, deg16_ref, srt_ref, ebuf, kdst, ksrc, h16, sortl, sem):
    t = _wid()
    lo = t * _NPT
    hi = lo + _NPT
    zeros = jnp.zeros((16,), jnp.int32)
    lane = lax.iota(jnp.int32, 16)

    def _z(i, _):
        h16[i, pl.ds(0, 16)] = zeros
        return 0
    lax.fori_loop(0, _NPT, _z, 0)

    def _z2(i, _):
        sortl[pl.ds(i * 16, 16)] = zeros
        return 0
    lax.fori_loop(0, _CAP // 16, _z2, 0)

    def _start(c, k):
        pltpu.async_copy(edge_ref.at[:, pl.ds(c * _CH, _CH)], ebuf.at[k], sem)

    def _waitc(k):
        pltpu.make_async_copy(edge_ref.at[:, pl.ds(0, _CH)], ebuf.at[k],
                              sem).wait()

    _start(0, 0)

    def _chunkpair(cb, cnt):
        for k in range(2):
            c = cb * 2 + k

            @pl.when(c + 1 < _NCH)
            def _():
                _start(c + 1, 1 - k)

            _waitc(k)

            def _vec(v, cnt):
                d = ebuf[k, 1, pl.ds(v * 16, 16)]
                m = (d >= lo) & (d < hi)
                rel = d - lo
                s = ebuf[k, 0, pl.ds(v * 16, 16)]
                w = jnp.minimum(cnt, _CAP - 16)
                rank = plsc.cumsum(m.astype(jnp.int32))
                pos = jnp.clip(w + rank - 1, 0, _CAP - 1)
                plsc.store_scatter(kdst, [pos], rel, mask=m)
                plsc.store_scatter(ksrc, [pos], s, mask=m)
                return cnt + rank[15]
            cnt = lax.fori_loop(0, _CH // 16, _vec, cnt)
        return cnt

    cnt = lax.fori_loop(0, _NCH // 2, _chunkpair, jnp.int32(0))
    cnt = jnp.minimum(cnt, _CAP)
    nv = (cnt + 15) // 16

    def _hist(v, _):
        m = (v * 16 + lane) < cnt
        d = jnp.clip(kdst[pl.ds(v * 16, 16)], 0, _NPT - 1)
        c = plsc.load_gather(h16, [d, lane], mask=m)
        plsc.store_scatter(h16, [d, lane], c + 1, mask=m)
        return 0
    lax.fori_loop(0, nv, _hist, 0)

    pltpu.sync_copy(h16, deg16_ref.at[pl.ds(t * _NPT, _NPT)])

    def _cum(i, base):
        v = h16[i, pl.ds(0, 16)]
        inc = plsc.cumsum(v)
        h16[i, pl.ds(0, 16)] = inc - v + base
        return base + inc[15]
    lax.fori_loop(0, _NPT, _cum, jnp.int32(0))

    def _place(v, _):
        m = (v * 16 + lane) < cnt
        d = jnp.clip(kdst[pl.ds(v * 16, 16)], 0, _NPT - 1)
        s = ksrc[pl.ds(v * 16, 16)]
        p = plsc.load_gather(h16, [d, lane], mask=m)
        plsc.store_scatter(sortl, [jnp.clip(p, 0, _CAP - 1)], s, mask=m)
        plsc.store_scatter(h16, [d, lane], p + 1, mask=m)
        return 0
    lax.fori_loop(0, nv, _place, 0)

    pltpu.sync_copy(sortl, srt_ref.at[pl.ds(t * _CAP, _CAP)])


@functools.partial(
    pl.kernel,
    out_type=jax.ShapeDtypeStruct((_NPAD, 4 * _EMB), jnp.float32),
    mesh=_mesh,
    scratch_types=[
        pltpu.VMEM((_NPT + 16,), jnp.int32),
        pltpu.VMEM((_CAP,), jnp.int32),
        pltpu.VMEM((2, _GC), jnp.int32),
        pltpu.VMEM((2, _GC, _EMB), jnp.float32),
        pltpu.VMEM((64, 4 * _EMB), jnp.float32),
        pltpu.SemaphoreType.DMA,
    ],
    compiler_params=_sc_params,
)
def _sc_fold(b_ref, deg_ref, srt_ref, stats_ref, degw, srt, idxb, rows,
             outb, sem):
    t = _wid()
    zf = jnp.zeros((16,), jnp.float32)
    neg = jnp.full((16,), _NEG, jnp.float32)
    pos = jnp.full((16,), _POS, jnp.float32)

    pltpu.sync_copy(deg_ref.at[pl.ds(t * _NPT, _NPT)], degw.at[pl.ds(0, _NPT)])
    pltpu.sync_copy(srt_ref.at[pl.ds(t * _CAP, _CAP)], srt)

    def _deg(n):
        return degw[pl.ds(jnp.minimum(n, _NPT - 1), 16)][0]

    def _sum(i, acc):
        return acc + degw[pl.ds(i * 16, 16)]
    tot = lax.fori_loop(0, _NPT // 16, _sum, jnp.zeros((16,), jnp.int32))
    count = jnp.minimum(plsc.cumsum(tot)[15], _CAP)
    nch = (count + _GC - 1) // _GC

    def _fire(c, k):
        def _cp(i, _):
            idxb[k, pl.ds(i * 16, 16)] = srt[pl.ds(c * _GC + i * 16, 16)]
            return 0
        lax.fori_loop(0, _GC // 16, _cp, 0)
        pltpu.async_copy(b_ref.at[idxb.at[k]], rows.at[k], sem)

    def _waitg(k):
        pltpu.make_async_copy(b_ref.at[idxb.at[k]], rows.at[k], sem).wait()

    def _flush_adv(state):
        n, r, s10, s11, s12, q0, q1, q2, x0, x1, x2, m0, m1, m2 = state
        row = n & 63
        outb[row, pl.ds(0, 16)] = s10
        outb[row, pl.ds(16, 16)] = s11
        outb[row, pl.ds(32, 16)] = s12
        outb[row, pl.ds(48, 16)] = q0
        outb[row, pl.ds(64, 16)] = q1
        outb[row, pl.ds(80, 16)] = q2
        outb[row, pl.ds(96, 16)] = x0
        outb[row, pl.ds(112, 16)] = x1
        outb[row, pl.ds(128, 16)] = x2
        outb[row, pl.ds(144, 16)] = m0
        outb[row, pl.ds(160, 16)] = m1
        outb[row, pl.ds(176, 16)] = m2

        @pl.when(row == 63)
        def _():
            pltpu.sync_copy(outb, stats_ref.at[pl.ds(t * _NPT + (n - 63), 64)])

        n = n + 1
        r = jnp.where(n < _NPT, _deg(n), jnp.int32(1))
        return (n, r, zf, zf, zf, zf, zf, zf, neg, neg, neg, pos, pos, pos)

    def _skip_empty(state):
        return lax.while_loop(
            lambda st: (st[1] == 0) & (st[0] < _NPT), _flush_adv, state)

    state = (jnp.int32(0), _deg(jnp.int32(0)), zf, zf, zf, zf, zf, zf,
             neg, neg, neg, pos, pos, pos)
    state = _skip_empty(state)

    @pl.when(nch > 0)
    def _():
        _fire(0, 0)

    def _chunkpair(cb, state):
        for k in range(2):
            c = cb * 2 + k

            @pl.when(c + 1 < nch)
            def _():
                _fire(c + 1, 1 - k)

            @pl.when(c < nch)
            def _():
                _waitg(k)

            def _edge(i, st):
                n, r, s10, s11, s12, q0, q1, q2, x0, x1, x2, m0, m1, m2 = st
                v0 = rows[k, i, pl.ds(0, 16)]
                v1 = rows[k, i, pl.ds(16, 16)]
                v2 = rows[k, i, pl.ds(32, 16)]
                st = (n, r - 1,
                      s10 + v0, s11 + v1, s12 + v2,
                      q0 + v0 * v0, q1 + v1 * v1, q2 + v2 * v2,
                      jnp.maximum(x0, v0), jnp.maximum(x1, v1),
                      jnp.maximum(x2, v2),
                      jnp.minimum(m0, v0), jnp.minimum(m1, v1),
                      jnp.minimum(m2, v2))
                return _skip_empty(st)

            ne = jnp.clip(count - c * _GC, 0, _GC)
            state = lax.fori_loop(0, ne, _edge, state)
        return state

    lax.fori_loop(0, (nch + 1) // 2, _chunkpair, state)


def _mm_body(x_ref, w_ref, b_ref, o_ref):
    o_ref[...] = (
        jnp.dot(x_ref[...], w_ref[...], preferred_element_type=jnp.float32)
        + b_ref[...]
    )


def _mm(x, w, b):
    n, k = x.shape
    m = w.shape[1]
    return pl.pallas_call(
        _mm_body,
        grid=(n // _BR,),
        in_specs=[
            pl.BlockSpec((_BR, k), lambda i: (i, 0)),
            pl.BlockSpec((k, m), lambda i: (0, 0)),
            pl.BlockSpec((1, m), lambda i: (0, 0)),
        ],
        out_specs=pl.BlockSpec((_BR, m), lambda i: (i, 0)),
        out_shape=jax.ShapeDtypeStruct((n, m), jnp.float32),
    )(x, w, b.reshape(1, m))


def _blockdiag(w):
    t, k, f = w.shape
    out = jnp.zeros((t, k, t, f), jnp.float32)
    for i in range(t):
        out = out.at[i, :, i, :].set(w[i])
    return out.reshape(t * k, t * f)


def kernel(x, edge_index, enc_W, enc_b, pre_W, pre_b, post_W, post_b,
           lin_W, lin_b, bn_gamma, bn_beta):
    deg16, srt = _sc_group(edge_index)
    deg_i = deg16.sum(axis=1)
    deg = deg_i[:_N].astype(jnp.float32)
    deg_c = jnp.maximum(deg, 1.0)
    log_deg = jnp.log(deg_c + 1.0)
    has = (deg > 0)[:, None]
    amp = (log_deg / _AVG_LOG)[:, None]
    att = (_AVG_LOG / log_deg)[:, None]
    dcv = deg_c[:, None]

    h = _mm(x, enc_W, enc_b)
    for l in range(_L):
        Wd = _blockdiag(pre_W[l, :, :_F, :])
        Ws = _blockdiag(pre_W[l, :, _F:, :])
        A = _mm(h, Wd, pre_b[l].reshape(_EMB))
        B = _mm(h, Ws, jnp.zeros((_EMB,), jnp.float32))
        stats = _sc_fold(B, deg_i, srt)[:_N]
        S1 = stats[:, 0:48]
        S2 = stats[:, 48:96]
        Mx = stats[:, 96:144]
        Mn = stats[:, 144:192]
        mean = jnp.where(has, A + S1 / dcv, 0.0)
        std = jnp.sqrt(jax.nn.relu(S2 / dcv - (S1 / dcv) ** 2) + 1e-5)
        mx = jnp.where(has, A + Mx, 0.0)
        mn = jnp.where(has, A + Mn, 0.0)
        feat = jnp.concatenate([
            h, mean, mx, mn, std,
            mean * amp, mx * amp, mn * amp, std * amp,
            mean * att, mx * att, mn * att, std * att,
        ], axis=1)
        Wp = _blockdiag(post_W[l])
        Wp = Wp.reshape(_T, 13, _F, _EMB).transpose(1, 0, 2, 3).reshape(624, _EMB)
        Wfull = Wp @ lin_W[l]
        bfull = post_b[l].reshape(_EMB) @ lin_W[l] + lin_b[l]
        out = _mm(feat, Wfull, bfull)
        mu = out.mean(axis=0)
        var = out.var(axis=0)
        out = (out - mu) / jnp.sqrt(var + 1e-5) * bn_gamma[l] + bn_beta[l]
        h = jax.nn.relu(out) + h
    return h


# speedup vs baseline: 217.4258x; 1.2405x over previous
import functools
import math

import jax
import jax.numpy as jnp
from jax import lax
from jax.experimental import pallas as pl
from jax.experimental.pallas import tpu as pltpu
from jax.experimental.pallas import tpu_sc as plsc

_N = 50000
_E = 800000
_D_IN = 128
_EMB = 48
_T = 4
_F = 12
_L = 4
_AVG_LOG = float(math.log(17.0))
_BR = 1000

_NC = 2
_NS = 16
_NW = _NC * _NS
_NPT = 1600
_NPAD = _NW * _NPT
_CAP = 30720
_CH = 2000
_NCH = _E // _CH
_GC = 128

_mesh = plsc.VectorSubcoreMesh(core_axis_name="c", subcore_axis_name="s")
_sc_params = pltpu.CompilerParams(needs_layout_passes=False,
                                  use_tc_tiling_on_sc=False)

_NEG = float("-inf")
_POS = float("inf")
_RB = 1


def _wid():
    return lax.axis_index("s") * _NC + lax.axis_index("c")


@functools.partial(
    pl.kernel,
    out_type=(
        jax.ShapeDtypeStruct((_NPAD, 16), jnp.int32),
        jax.ShapeDtypeStruct((_NW * _CAP,), jnp.int32),
    ),
    mesh=_mesh,
    scratch_types=[
        pltpu.VMEM((2, 2, _CH), jnp.int32),
        pltpu.VMEM((_CAP,), jnp.int32),
        pltpu.VMEM((_CAP,), jnp.int32),
        pltpu.VMEM((_NPT, 16), jnp.int32),
        pltpu.VMEM((_CAP,), jnp.int32),
        pltpu.SemaphoreType.DMA,
    ],
    compiler_params=_sc_params,
)
def _sc_group(edge_ref, deg16_ref, srt_ref, ebuf, kdst, ksrc, h16, sortl, sem):
    t = _wid()
    lo = t * _NPT
    hi = lo + _NPT
    zeros = jnp.zeros((16,), jnp.int32)
    lane = lax.iota(jnp.int32, 16)

    def _z(i, _):
        h16[i, pl.ds(0, 16)] = zeros
        return 0
    lax.fori_loop(0, _NPT, _z, 0)

    def _z2(i, _):
        sortl[pl.ds(i * 16, 16)] = zeros
        return 0
    lax.fori_loop(0, _CAP // 16, _z2, 0)

    def _start(c, k):
        pltpu.async_copy(edge_ref.at[:, pl.ds(c * _CH, _CH)], ebuf.at[k], sem)

    def _waitc(k):
        pltpu.make_async_copy(edge_ref.at[:, pl.ds(0, _CH)], ebuf.at[k],
                              sem).wait()

    _start(0, 0)

    def _chunkpair(cb, cnt):
        for k in range(2):
            c = cb * 2 + k

            @pl.when(c + 1 < _NCH)
            def _():
                _start(c + 1, 1 - k)

            _waitc(k)

            def _vec(v, cnt):
                d = ebuf[k, 1, pl.ds(v * 16, 16)]
                m = (d >= lo) & (d < hi)
                rel = d - lo
                s = ebuf[k, 0, pl.ds(v * 16, 16)]
                w = jnp.minimum(cnt, _CAP - 16)
                rank = plsc.cumsum(m.astype(jnp.int32))
                pos = jnp.clip(w + rank - 1, 0, _CAP - 1)
                plsc.store_scatter(kdst, [pos], rel, mask=m)
                plsc.store_scatter(ksrc, [pos], s, mask=m)
                return cnt + rank[15]
            cnt = lax.fori_loop(0, _CH // 16, _vec, cnt)
        return cnt

    cnt = lax.fori_loop(0, _NCH // 2, _chunkpair, jnp.int32(0))
    cnt = jnp.minimum(cnt, _CAP)
    nv = (cnt + 15) // 16

    def _hist(v, _):
        m = (v * 16 + lane) < cnt
        d = jnp.clip(kdst[pl.ds(v * 16, 16)], 0, _NPT - 1)
        c = plsc.load_gather(h16, [d, lane], mask=m)
        plsc.store_scatter(h16, [d, lane], c + 1, mask=m)
        return 0
    lax.fori_loop(0, nv, _hist, 0)

    pltpu.sync_copy(h16, deg16_ref.at[pl.ds(t * _NPT, _NPT)])

    def _cum(i, base):
        v = h16[i, pl.ds(0, 16)]
        inc = plsc.cumsum(v)
        h16[i, pl.ds(0, 16)] = inc - v + base
        return base + inc[15]
    lax.fori_loop(0, _NPT, _cum, jnp.int32(0))

    def _place(v, _):
        m = (v * 16 + lane) < cnt
        d = jnp.clip(kdst[pl.ds(v * 16, 16)], 0, _NPT - 1)
        s = ksrc[pl.ds(v * 16, 16)]
        p = plsc.load_gather(h16, [d, lane], mask=m)
        plsc.store_scatter(sortl, [jnp.clip(p, 0, _CAP - 1)], s, mask=m)
        plsc.store_scatter(h16, [d, lane], p + 1, mask=m)
        return 0
    lax.fori_loop(0, nv, _place, 0)

    pltpu.sync_copy(sortl, srt_ref.at[pl.ds(t * _CAP, _CAP)])


@functools.partial(
    pl.kernel,
    out_type=jax.ShapeDtypeStruct((_NPAD, 4 * _EMB), jnp.float32),
    mesh=_mesh,
    scratch_types=[
        pltpu.VMEM((_NPT + 16,), jnp.int32),
        pltpu.VMEM((_CAP,), jnp.int32),
        pltpu.VMEM((2, _GC), jnp.int32),
        pltpu.VMEM((2, _GC, _EMB), jnp.float32),
        pltpu.VMEM((64, 4 * _EMB), jnp.float32),
        pltpu.SemaphoreType.DMA,
    ],
    compiler_params=_sc_params,
)
def _sc_fold(b_ref, deg_ref, srt_ref, stats_ref, degw, srt, idxb, rows,
             outb, sem):
    t = _wid()
    zf = jnp.zeros((16,), jnp.float32)
    neg = jnp.full((16,), _NEG, jnp.float32)
    pos = jnp.full((16,), _POS, jnp.float32)

    pltpu.sync_copy(deg_ref.at[pl.ds(t * _NPT, _NPT)], degw.at[pl.ds(0, _NPT)])
    pltpu.sync_copy(srt_ref.at[pl.ds(t * _CAP, _CAP)], srt)

    def _deg(n):
        return degw[pl.ds(jnp.minimum(n, _NPT - 1), 16)][0]

    def _sum(i, acc):
        return acc + degw[pl.ds(i * 16, 16)]
    tot = lax.fori_loop(0, _NPT // 16, _sum, jnp.zeros((16,), jnp.int32))
    count = jnp.minimum(plsc.cumsum(tot)[15], _CAP)
    nch = (count + _GC - 1) // _GC

    def _fire(c, k):
        def _cp(i, _):
            idxb[k, pl.ds(i * 16, 16)] = srt[pl.ds(c * _GC + i * 16, 16)]
            return 0
        lax.fori_loop(0, _GC // 16, _cp, 0)
        pltpu.async_copy(b_ref.at[idxb.at[k]], rows.at[k], sem)

    def _waitg(k):
        pltpu.make_async_copy(b_ref.at[idxb.at[k]], rows.at[k], sem).wait()

    def _flush_adv(state):
        n, r, s10, s11, s12, q0, q1, q2, x0, x1, x2, m0, m1, m2 = state
        row = n & 63
        outb[row, pl.ds(0, 16)] = s10
        outb[row, pl.ds(16, 16)] = s11
        outb[row, pl.ds(32, 16)] = s12
        outb[row, pl.ds(48, 16)] = q0
        outb[row, pl.ds(64, 16)] = q1
        outb[row, pl.ds(80, 16)] = q2
        outb[row, pl.ds(96, 16)] = x0
        outb[row, pl.ds(112, 16)] = x1
        outb[row, pl.ds(128, 16)] = x2
        outb[row, pl.ds(144, 16)] = m0
        outb[row, pl.ds(160, 16)] = m1
        outb[row, pl.ds(176, 16)] = m2

        @pl.when(row == 63)
        def _():
            pltpu.sync_copy(outb, stats_ref.at[pl.ds(t * _NPT + (n - 63), 64)])

        n = n + 1
        r = jnp.where(n < _NPT, _deg(n), jnp.int32(1))
        return (n, r, zf, zf, zf, zf, zf, zf, neg, neg, neg, pos, pos, pos)

    def _skip_empty(state):
        return lax.while_loop(
            lambda st: (st[1] == 0) & (st[0] < _NPT), _flush_adv, state)

    state = (jnp.int32(0), _deg(jnp.int32(0)), zf, zf, zf, zf, zf, zf,
             neg, neg, neg, pos, pos, pos)
    state = _skip_empty(state)

    @pl.when(nch > 0)
    def _():
        _fire(0, 0)

    def _chunkpair(cb, state):
        for k in range(2):
            c = cb * 2 + k

            @pl.when(c + 1 < nch)
            def _():
                _fire(c + 1, 1 - k)

            @pl.when(c < nch)
            def _():
                _waitg(k)

            def _edge(i, st):
                n, r, s10, s11, s12, q0, q1, q2, x0, x1, x2, m0, m1, m2 = st
                v0 = rows[k, i, pl.ds(0, 16)]
                v1 = rows[k, i, pl.ds(16, 16)]
                v2 = rows[k, i, pl.ds(32, 16)]
                st = (n, r - 1,
                      s10 + v0, s11 + v1, s12 + v2,
                      q0 + v0 * v0, q1 + v1 * v1, q2 + v2 * v2,
                      jnp.maximum(x0, v0), jnp.maximum(x1, v1),
                      jnp.maximum(x2, v2),
                      jnp.minimum(m0, v0), jnp.minimum(m1, v1),
                      jnp.minimum(m2, v2))
                return _skip_empty(st)

            ne = jnp.clip(count - c * _GC, 0, _GC)
            state = lax.fori_loop(0, ne, _edge, state)
        return state

    lax.fori_loop(0, (nch + 1) // 2, _chunkpair, state)


def _combine_body(h_ref, st_ref, sc_ref, wd_ref, pb_ref, wh_ref, vc_ref,
                  bf_ref, o_ref):
    h = h_ref[...]
    st = st_ref[...]
    inv = sc_ref[:, 0:1]
    hb = sc_ref[:, 1:2] > 0.0
    amp = sc_ref[:, 2:3]
    att = sc_ref[:, 3:4]
    A = jnp.dot(h, wd_ref[...], preferred_element_type=jnp.float32) + pb_ref[...]
    S1 = st[:, 0:48]
    S2 = st[:, 48:96]
    Mx = st[:, 96:144]
    Mn = st[:, 144:192]
    e1 = S1 * inv
    mean = jnp.where(hb, A + e1, 0.0)
    std = jnp.sqrt(jax.nn.relu(S2 * inv - e1 * e1) + 1e-5)
    mx = jnp.where(hb, A + Mx, 0.0)
    mn = jnp.where(hb, A + Mn, 0.0)
    M = jnp.concatenate([mean, mx, mn, std], axis=1)
    P = jnp.dot(M, vc_ref[...], preferred_element_type=jnp.float32)
    o_ref[...] = (
        jnp.dot(h, wh_ref[...], preferred_element_type=jnp.float32)
        + P[:, 0:48] + amp * P[:, 48:96] + att * P[:, 96:144] + bf_ref[...]
    )


def _combine(h, stats, scal, Wd, pre_b, Wh, Vcat, bfull):
    n = h.shape[0]
    return pl.pallas_call(
        _combine_body,
        grid=(n // _BR,),
        in_specs=[
            pl.BlockSpec((_BR, _EMB), lambda i: (i, 0)),
            pl.BlockSpec((_BR, 4 * _EMB), lambda i: (i, 0)),
            pl.BlockSpec((_BR, 4), lambda i: (i, 0)),
            pl.BlockSpec((_EMB, _EMB), lambda i: (0, 0)),
            pl.BlockSpec((1, _EMB), lambda i: (0, 0)),
            pl.BlockSpec((_EMB, _EMB), lambda i: (0, 0)),
            pl.BlockSpec((4 * _EMB, 3 * _EMB), lambda i: (0, 0)),
            pl.BlockSpec((1, _EMB), lambda i: (0, 0)),
        ],
        out_specs=pl.BlockSpec((_BR, _EMB), lambda i: (i, 0)),
        out_shape=jax.ShapeDtypeStruct((n, _EMB), jnp.float32),
    )(h, stats, scal, Wd, pre_b.reshape(1, _EMB), Wh, Vcat,
      bfull.reshape(1, _EMB))


def _mm_body(x_ref, w_ref, b_ref, o_ref):
    o_ref[...] = (
        jnp.dot(x_ref[...], w_ref[...], preferred_element_type=jnp.float32)
        + b_ref[...]
    )


def _mm(x, w, b):
    n, k = x.shape
    m = w.shape[1]
    return pl.pallas_call(
        _mm_body,
        grid=(n // _BR,),
        in_specs=[
            pl.BlockSpec((_BR, k), lambda i: (i, 0)),
            pl.BlockSpec((k, m), lambda i: (0, 0)),
            pl.BlockSpec((1, m), lambda i: (0, 0)),
        ],
        out_specs=pl.BlockSpec((_BR, m), lambda i: (i, 0)),
        out_shape=jax.ShapeDtypeStruct((n, m), jnp.float32),
    )(x, w, b.reshape(1, m))


def _blockdiag(w):
    t, k, f = w.shape
    out = jnp.zeros((t, k, t, f), jnp.float32)
    for i in range(t):
        out = out.at[i, :, i, :].set(w[i])
    return out.reshape(t * k, t * f)


def kernel(x, edge_index, enc_W, enc_b, pre_W, pre_b, post_W, post_b,
           lin_W, lin_b, bn_gamma, bn_beta):
    deg16, srt = _sc_group(edge_index)
    deg_i = deg16.sum(axis=1)
    deg = deg_i[:_N].astype(jnp.float32)
    deg_c = jnp.maximum(deg, 1.0)
    log_deg = jnp.log(deg_c + 1.0)
    scal = jnp.stack([
        1.0 / deg_c,
        (deg > 0).astype(jnp.float32),
        log_deg / _AVG_LOG,
        _AVG_LOG / log_deg,
    ], axis=1)

    h = _mm(x, enc_W, enc_b)
    for l in range(_L):
        Wd = _blockdiag(pre_W[l, :, :_F, :])
        Ws = _blockdiag(pre_W[l, :, _F:, :])
        B = _mm(h, Ws, jnp.zeros((_EMB,), jnp.float32))
        stats = _sc_fold(B, deg_i, srt)
        Wp = _blockdiag(post_W[l])
        Wp = Wp.reshape(_T, 13, _F, _EMB).transpose(1, 0, 2, 3).reshape(624, _EMB)
        Wfull = Wp @ lin_W[l]
        bfull = post_b[l].reshape(_EMB) @ lin_W[l] + lin_b[l]
        Wh = Wfull[0:_EMB]
        Vcat = jnp.concatenate(
            [Wfull[_EMB:5 * _EMB], Wfull[5 * _EMB:9 * _EMB],
             Wfull[9 * _EMB:13 * _EMB]], axis=1)
        out = _combine(h, stats, scal, Wd, pre_b[l].reshape(_EMB), Wh, Vcat,
                       bfull)
        mu = out.mean(axis=0)
        var = out.var(axis=0)
        out = (out - mu) / jnp.sqrt(var + 1e-5) * bn_gamma[l] + bn_beta[l]
        h = jax.nn.relu(out) + h
    return h


# speedup vs baseline: 471.2657x; 2.1675x over previous
import functools
import math

import jax
import jax.numpy as jnp
from jax import lax
from jax.experimental import pallas as pl
from jax.experimental.pallas import tpu as pltpu
from jax.experimental.pallas import tpu_sc as plsc

_N = 50000
_E = 800000
_D_IN = 128
_EMB = 48
_T = 4
_F = 12
_L = 4
_AVG_LOG = float(math.log(17.0))
_BR = 1000

_NC = 2
_NS = 16
_NW = _NC * _NS
_NPT = 1600
_NPAD = _NW * _NPT
_CAP = 30720
_CH = 2000
_NCH = _E // _CH
_GC = 128

_mesh = plsc.VectorSubcoreMesh(core_axis_name="c", subcore_axis_name="s")
_sc_params = pltpu.CompilerParams(needs_layout_passes=False,
                                  use_tc_tiling_on_sc=False)

_NEG = float("-inf")
_POS = float("inf")
_RB = 1


def _wid():
    return lax.axis_index("s") * _NC + lax.axis_index("c")


@functools.partial(
    pl.kernel,
    out_type=(
        jax.ShapeDtypeStruct((_NPAD, 16), jnp.int32),
        jax.ShapeDtypeStruct((_NW * _CAP,), jnp.int32),
    ),
    mesh=_mesh,
    scratch_types=[
        pltpu.VMEM((2, 2, _CH), jnp.int32),
        pltpu.VMEM((_CAP,), jnp.int32),
        pltpu.VMEM((_CAP,), jnp.int32),
        pltpu.VMEM((_NPT, 16), jnp.int32),
        pltpu.VMEM((_CAP,), jnp.int32),
        pltpu.SemaphoreType.DMA,
    ],
    compiler_params=_sc_params,
)
def _sc_group(edge_ref, deg16_ref, srt_ref, ebuf, kdst, ksrc, h16, sortl, sem):
    t = _wid()
    lo = t * _NPT
    hi = lo + _NPT
    zeros = jnp.zeros((16,), jnp.int32)
    lane = lax.iota(jnp.int32, 16)

    def _z(i, _):
        h16[i, pl.ds(0, 16)] = zeros
        return 0
    lax.fori_loop(0, _NPT, _z, 0)

    def _z2(i, _):
        sortl[pl.ds(i * 16, 16)] = zeros
        return 0
    lax.fori_loop(0, _CAP // 16, _z2, 0)

    def _start(c, k):
        pltpu.async_copy(edge_ref.at[:, pl.ds(c * _CH, _CH)], ebuf.at[k], sem)

    def _waitc(k):
        pltpu.make_async_copy(edge_ref.at[:, pl.ds(0, _CH)], ebuf.at[k],
                              sem).wait()

    _start(0, 0)

    def _chunkpair(cb, cnt):
        for k in range(2):
            c = cb * 2 + k

            @pl.when(c + 1 < _NCH)
            def _():
                _start(c + 1, 1 - k)

            _waitc(k)

            def _vec(v, cnt):
                d = ebuf[k, 1, pl.ds(v * 16, 16)]
                m = (d >= lo) & (d < hi)
                rel = d - lo
                s = ebuf[k, 0, pl.ds(v * 16, 16)]
                w = jnp.minimum(cnt, _CAP - 16)
                rank = plsc.cumsum(m.astype(jnp.int32))
                pos = jnp.clip(w + rank - 1, 0, _CAP - 1)
                plsc.store_scatter(kdst, [pos], rel, mask=m)
                plsc.store_scatter(ksrc, [pos], s, mask=m)
                return cnt + rank[15]
            cnt = lax.fori_loop(0, _CH // 16, _vec, cnt)
        return cnt

    cnt = lax.fori_loop(0, _NCH // 2, _chunkpair, jnp.int32(0))
    cnt = jnp.minimum(cnt, _CAP)
    nv = (cnt + 15) // 16

    def _hist(v, _):
        m = (v * 16 + lane) < cnt
        d = jnp.clip(kdst[pl.ds(v * 16, 16)], 0, _NPT - 1)
        c = plsc.load_gather(h16, [d, lane], mask=m)
        plsc.store_scatter(h16, [d, lane], c + 1, mask=m)
        return 0
    lax.fori_loop(0, nv, _hist, 0)

    pltpu.sync_copy(h16, deg16_ref.at[pl.ds(t * _NPT, _NPT)])

    def _cum(i, base):
        v = h16[i, pl.ds(0, 16)]
        inc = plsc.cumsum(v)
        h16[i, pl.ds(0, 16)] = inc - v + base
        return base + inc[15]
    lax.fori_loop(0, _NPT, _cum, jnp.int32(0))

    def _place(v, _):
        m = (v * 16 + lane) < cnt
        d = jnp.clip(kdst[pl.ds(v * 16, 16)], 0, _NPT - 1)
        s = ksrc[pl.ds(v * 16, 16)]
        p = plsc.load_gather(h16, [d, lane], mask=m)
        plsc.store_scatter(sortl, [jnp.clip(p, 0, _CAP - 1)], s, mask=m)
        plsc.store_scatter(h16, [d, lane], p + 1, mask=m)
        return 0
    lax.fori_loop(0, nv, _place, 0)

    pltpu.sync_copy(sortl, srt_ref.at[pl.ds(t * _CAP, _CAP)])


@functools.partial(
    pl.kernel,
    out_type=jax.ShapeDtypeStruct((_NPAD, 4 * _EMB), jnp.float32),
    mesh=_mesh,
    scratch_types=[
        pltpu.VMEM((_NPT + 16,), jnp.int32),
        pltpu.VMEM((_CAP,), jnp.int32),
        pltpu.VMEM((2, _GC), jnp.int32),
        pltpu.VMEM((2, _GC, _EMB), jnp.float32),
        pltpu.VMEM((64, 4 * _EMB), jnp.float32),
        pltpu.SemaphoreType.DMA,
    ],
    compiler_params=_sc_params,
)
def _sc_fold(b_ref, deg_ref, srt_ref, stats_ref, degw, srt, idxb, rows,
             outb, sem):
    t = _wid()
    zf = jnp.zeros((16,), jnp.float32)
    neg = jnp.full((16,), _NEG, jnp.float32)
    pos = jnp.full((16,), _POS, jnp.float32)

    pltpu.sync_copy(deg_ref.at[pl.ds(t * _NPT, _NPT)], degw.at[pl.ds(0, _NPT)])
    pltpu.sync_copy(srt_ref.at[pl.ds(t * _CAP, _CAP)], srt)

    def _deg(n):
        return degw[pl.ds(jnp.minimum(n, _NPT - 1), 16)][0]

    def _sum(i, acc):
        return acc + degw[pl.ds(i * 16, 16)]
    tot = lax.fori_loop(0, _NPT // 16, _sum, jnp.zeros((16,), jnp.int32))
    count = jnp.minimum(plsc.cumsum(tot)[15], _CAP)
    nch = (count + _GC - 1) // _GC

    def _fire(c, k):
        def _cp(i, _):
            idxb[k, pl.ds(i * 16, 16)] = srt[pl.ds(c * _GC + i * 16, 16)]
            return 0
        lax.fori_loop(0, _GC // 16, _cp, 0)
        pltpu.async_copy(b_ref.at[idxb.at[k]], rows.at[k], sem)

    def _waitg(k):
        pltpu.make_async_copy(b_ref.at[idxb.at[k]], rows.at[k], sem).wait()

    def _flush_adv(state):
        n, r, s10, s11, s12, q0, q1, q2, x0, x1, x2, m0, m1, m2 = state
        row = n & 63
        outb[row, pl.ds(0, 16)] = s10
        outb[row, pl.ds(16, 16)] = s11
        outb[row, pl.ds(32, 16)] = s12
        outb[row, pl.ds(48, 16)] = q0
        outb[row, pl.ds(64, 16)] = q1
        outb[row, pl.ds(80, 16)] = q2
        outb[row, pl.ds(96, 16)] = x0
        outb[row, pl.ds(112, 16)] = x1
        outb[row, pl.ds(128, 16)] = x2
        outb[row, pl.ds(144, 16)] = m0
        outb[row, pl.ds(160, 16)] = m1
        outb[row, pl.ds(176, 16)] = m2

        @pl.when(row == 63)
        def _():
            pltpu.sync_copy(outb, stats_ref.at[pl.ds(t * _NPT + (n - 63), 64)])

        n = n + 1
        r = jnp.where(n < _NPT, _deg(n), jnp.int32(1))
        return (n, r, zf, zf, zf, zf, zf, zf, neg, neg, neg, pos, pos, pos)

    def _skip_empty(state):
        return lax.while_loop(
            lambda st: (st[1] == 0) & (st[0] < _NPT), _flush_adv, state)

    state = (jnp.int32(0), _deg(jnp.int32(0)), zf, zf, zf, zf, zf, zf,
             neg, neg, neg, pos, pos, pos)
    state = _skip_empty(state)

    @pl.when(nch > 0)
    def _():
        _fire(0, 0)

    def _chunkpair(cb, state):
        for k in range(2):
            c = cb * 2 + k

            @pl.when(c + 1 < nch)
            def _():
                _fire(c + 1, 1 - k)

            @pl.when(c < nch)
            def _():
                _waitg(k)

            ne = jnp.clip(count - c * _GC, 0, _GC)

            def _run(st):
                n, r, p = st[0], st[1], st[2]
                take = jnp.minimum(r, ne - p)

                def _acc(j, a):
                    s10, s11, s12, q0, q1, q2, x0, x1, x2, m0, m1, m2 = a
                    v0 = rows[k, p + j, pl.ds(0, 16)]
                    v1 = rows[k, p + j, pl.ds(16, 16)]
                    v2 = rows[k, p + j, pl.ds(32, 16)]
                    return (s10 + v0, s11 + v1, s12 + v2,
                            q0 + v0 * v0, q1 + v1 * v1, q2 + v2 * v2,
                            jnp.maximum(x0, v0), jnp.maximum(x1, v1),
                            jnp.maximum(x2, v2),
                            jnp.minimum(m0, v0), jnp.minimum(m1, v1),
                            jnp.minimum(m2, v2))

                accs = lax.fori_loop(0, take, _acc, st[3:])
                st2 = _skip_empty((n, r - take) + accs)
                return (st2[0], st2[1], p + take) + st2[2:]

            st14 = (state[0], state[1], jnp.int32(0)) + state[2:]
            st14 = lax.while_loop(lambda s: s[2] < ne, _run, st14)
            state = (st14[0], st14[1]) + st14[3:]
        return state

    lax.fori_loop(0, (nch + 1) // 2, _chunkpair, state)


def _combine_body(h_ref, st_ref, sc_ref, wd_ref, pb_ref, wh_ref, vc_ref,
                  bf_ref, o_ref):
    h = h_ref[...]
    st = st_ref[...]
    inv = sc_ref[:, 0:1]
    hb = sc_ref[:, 1:2] > 0.0
    amp = sc_ref[:, 2:3]
    att = sc_ref[:, 3:4]
    A = jnp.dot(h, wd_ref[...], preferred_element_type=jnp.float32) + pb_ref[...]
    S1 = st[:, 0:48]
    S2 = st[:, 48:96]
    Mx = st[:, 96:144]
    Mn = st[:, 144:192]
    e1 = S1 * inv
    mean = jnp.where(hb, A + e1, 0.0)
    std = jnp.sqrt(jax.nn.relu(S2 * inv - e1 * e1) + 1e-5)
    mx = jnp.where(hb, A + Mx, 0.0)
    mn = jnp.where(hb, A + Mn, 0.0)
    M = jnp.concatenate([mean, mx, mn, std], axis=1)
    P = jnp.dot(M, vc_ref[...], preferred_element_type=jnp.float32)
    o_ref[...] = (
        jnp.dot(h, wh_ref[...], preferred_element_type=jnp.float32)
        + P[:, 0:48] + amp * P[:, 48:96] + att * P[:, 96:144] + bf_ref[...]
    )


def _combine(h, stats, scal, Wd, pre_b, Wh, Vcat, bfull):
    n = h.shape[0]
    return pl.pallas_call(
        _combine_body,
        grid=(n // _BR,),
        in_specs=[
            pl.BlockSpec((_BR, _EMB), lambda i: (i, 0)),
            pl.BlockSpec((_BR, 4 * _EMB), lambda i: (i, 0)),
            pl.BlockSpec((_BR, 4), lambda i: (i, 0)),
            pl.BlockSpec((_EMB, _EMB), lambda i: (0, 0)),
            pl.BlockSpec((1, _EMB), lambda i: (0, 0)),
            pl.BlockSpec((_EMB, _EMB), lambda i: (0, 0)),
            pl.BlockSpec((4 * _EMB, 3 * _EMB), lambda i: (0, 0)),
            pl.BlockSpec((1, _EMB), lambda i: (0, 0)),
        ],
        out_specs=pl.BlockSpec((_BR, _EMB), lambda i: (i, 0)),
        out_shape=jax.ShapeDtypeStruct((n, _EMB), jnp.float32),
    )(h, stats, scal, Wd, pre_b.reshape(1, _EMB), Wh, Vcat,
      bfull.reshape(1, _EMB))


def _mm_body(x_ref, w_ref, b_ref, o_ref):
    o_ref[...] = (
        jnp.dot(x_ref[...], w_ref[...], preferred_element_type=jnp.float32)
        + b_ref[...]
    )


def _mm(x, w, b):
    n, k = x.shape
    m = w.shape[1]
    return pl.pallas_call(
        _mm_body,
        grid=(n // _BR,),
        in_specs=[
            pl.BlockSpec((_BR, k), lambda i: (i, 0)),
            pl.BlockSpec((k, m), lambda i: (0, 0)),
            pl.BlockSpec((1, m), lambda i: (0, 0)),
        ],
        out_specs=pl.BlockSpec((_BR, m), lambda i: (i, 0)),
        out_shape=jax.ShapeDtypeStruct((n, m), jnp.float32),
    )(x, w, b.reshape(1, m))


def _blockdiag(w):
    t, k, f = w.shape
    out = jnp.zeros((t, k, t, f), jnp.float32)
    for i in range(t):
        out = out.at[i, :, i, :].set(w[i])
    return out.reshape(t * k, t * f)


def kernel(x, edge_index, enc_W, enc_b, pre_W, pre_b, post_W, post_b,
           lin_W, lin_b, bn_gamma, bn_beta):
    deg16, srt = _sc_group(edge_index)
    deg_i = deg16.sum(axis=1)
    deg = deg_i[:_N].astype(jnp.float32)
    deg_c = jnp.maximum(deg, 1.0)
    log_deg = jnp.log(deg_c + 1.0)
    scal = jnp.stack([
        1.0 / deg_c,
        (deg > 0).astype(jnp.float32),
        log_deg / _AVG_LOG,
        _AVG_LOG / log_deg,
    ], axis=1)

    h = _mm(x, enc_W, enc_b)
    for l in range(_L):
        Wd = _blockdiag(pre_W[l, :, :_F, :])
        Ws = _blockdiag(pre_W[l, :, _F:, :])
        B = _mm(h, Ws, jnp.zeros((_EMB,), jnp.float32))
        stats = _sc_fold(B, deg_i, srt)
        Wp = _blockdiag(post_W[l])
        Wp = Wp.reshape(_T, 13, _F, _EMB).transpose(1, 0, 2, 3).reshape(624, _EMB)
        Wfull = Wp @ lin_W[l]
        bfull = post_b[l].reshape(_EMB) @ lin_W[l] + lin_b[l]
        Wh = Wfull[0:_EMB]
        Vcat = jnp.concatenate(
            [Wfull[_EMB:5 * _EMB], Wfull[5 * _EMB:9 * _EMB],
             Wfull[9 * _EMB:13 * _EMB]], axis=1)
        out = _combine(h, stats, scal, Wd, pre_b[l].reshape(_EMB), Wh, Vcat,
                       bfull)
        mu = out.mean(axis=0)
        var = out.var(axis=0)
        out = (out - mu) / jnp.sqrt(var + 1e-5) * bn_gamma[l] + bn_beta[l]
        h = jax.nn.relu(out) + h
    return h
